# baseline (speedup 1.0000x reference)
import functools

import jax
import jax.numpy as jnp
from jax import lax
from jax.experimental import pallas as pl
from jax.experimental.pallas import tpu as pltpu
from jax.experimental.pallas import tpu_sc as plsc

_D = 64


def _make_gather(n_rows: int, chunk: int):
  info = plsc.get_sparse_core_info()
  nc, ns = info.num_cores, info.num_subcores
  nw = nc * ns
  assert n_rows % (nw * chunk) == 0
  per_w = n_rows // nw
  n_chunks = per_w // chunk

  mesh = plsc.VectorSubcoreMesh(core_axis_name="c", subcore_axis_name="s")

  @functools.partial(
      pl.kernel,
      mesh=mesh,
      out_type=jax.ShapeDtypeStruct((n_rows, _D), jnp.float32),
      scratch_types=[
          pltpu.VMEM((chunk,), jnp.int32),
          pltpu.VMEM((chunk, _D), jnp.float32),
          pltpu.SemaphoreType.DMA,
      ],
      compiler_params=pltpu.CompilerParams(use_tc_tiling_on_sc=False),
  )
  def gather_kernel(idx_hbm, table_hbm, out_hbm, idx_v, rows_v, sem):
    wid = lax.axis_index("s") * nc + lax.axis_index("c")
    base = wid * per_w

    def body(i, carry):
      off = base + i * chunk
      pltpu.sync_copy(idx_hbm.at[pl.ds(off, chunk)], idx_v)
      pltpu.async_copy(table_hbm.at[idx_v], rows_v, sem).wait()
      pltpu.sync_copy(rows_v, out_hbm.at[pl.ds(off, chunk)])
      return carry

    lax.fori_loop(0, n_chunks, body, 0)

  return gather_kernel


def kernel(x, table):
  b, s = x.shape
  flat_idx = x.reshape(b * s).astype(jnp.int32)
  out = _make_gather(b * s, 128)(flat_idx, table)
  return out.reshape(b, s, _D)


# speedup vs baseline: 1.1903x; 1.1903x over previous
import functools

import jax
import jax.numpy as jnp
from jax import lax
from jax.experimental import pallas as pl
from jax.experimental.pallas import tpu as pltpu
from jax.experimental.pallas import tpu_sc as plsc

_D = 64
_NBUF = 2


def _make_gather(n_rows: int, chunk: int):
  info = plsc.get_sparse_core_info()
  nc, ns = info.num_cores, info.num_subcores
  nw = nc * ns
  assert n_rows % (nw * chunk * _NBUF) == 0
  per_w = n_rows // nw
  n_chunks = per_w // chunk

  mesh = plsc.VectorSubcoreMesh(core_axis_name="c", subcore_axis_name="s")

  @functools.partial(
      pl.kernel,
      mesh=mesh,
      out_type=jax.ShapeDtypeStruct((n_rows, _D), jnp.float32),
      scratch_types=[
          pltpu.VMEM((per_w,), jnp.int32),
          pltpu.VMEM((_NBUF, chunk, _D), jnp.float32),
          pltpu.SemaphoreType.DMA((_NBUF,)),
          pltpu.SemaphoreType.DMA((_NBUF,)),
      ],
      compiler_params=pltpu.CompilerParams(use_tc_tiling_on_sc=False),
  )
  def gather_kernel(idx_hbm, table_hbm, out_hbm, idx_v, rows_v, gsem, ssem):
    wid = lax.axis_index("s") * nc + lax.axis_index("c")
    base = wid * per_w

    pltpu.sync_copy(idx_hbm.at[pl.ds(base, per_w)], idx_v)

    def start_gather(i, b):
      pltpu.async_copy(
          table_hbm.at[idx_v.at[pl.ds(i * chunk, chunk)]],
          rows_v.at[b],
          gsem.at[b],
      )

    def wait_gather(i, b):
      pltpu.make_async_copy(
          table_hbm.at[idx_v.at[pl.ds(i * chunk, chunk)]],
          rows_v.at[b],
          gsem.at[b],
      ).wait()

    def start_store(i, b):
      pltpu.async_copy(
          rows_v.at[b],
          out_hbm.at[pl.ds(base + i * chunk, chunk)],
          ssem.at[b],
      )

    def wait_store(i, b):
      pltpu.make_async_copy(
          rows_v.at[b],
          out_hbm.at[pl.ds(base + i * chunk, chunk)],
          ssem.at[b],
      ).wait()

    for b in range(_NBUF):
      start_gather(b, b)

    def outer(g, carry):
      for b in range(_NBUF):
        i = g * _NBUF + b
        wait_gather(i, b)
        start_store(i, b)

        @pl.when(i + _NBUF < n_chunks)
        def _():
          wait_store(i, b)
          start_gather(i + _NBUF, b)

      return carry

    lax.fori_loop(0, n_chunks // _NBUF, outer, 0)

    for b in range(_NBUF):
      wait_store(n_chunks - _NBUF + b, b)

  return gather_kernel


def kernel(x, table):
  b, s = x.shape
  flat_idx = x.reshape(b * s).astype(jnp.int32)
  out = _make_gather(b * s, 512)(flat_idx, table)
  return out.reshape(b, s, _D)
